# baseline (speedup 1.0000x reference)
import math

import jax
import jax.numpy as jnp
import numpy as np
from jax.experimental import pallas as pl

MAX_LEN = 500
EMBED_DIM = 128
NSPLIT = 5


def _pe_table() -> np.ndarray:
    pe = np.zeros((MAX_LEN, EMBED_DIM), dtype=np.float32)
    position = np.arange(0, MAX_LEN, dtype=np.float32)[:, None]
    div_term = np.exp(
        np.arange(0, EMBED_DIM, 2, dtype=np.float32) * (-math.log(10000.0) / EMBED_DIM)
    )
    pe[:, 0::2] = np.sin(position * div_term)
    pe[:, 1::2] = np.cos(position * div_term)
    return pe


def _body(*refs):
    mask_refs = refs[:NSPLIT]
    emb_ref, pe_ref, out_ref = refs[NSPLIT:]
    bb, lc, f = mask_refs[0].shape
    d = out_ref.shape[-1]
    e0 = emb_ref[0, :]
    de = (emb_ref[1, :] - e0) * (1.0 / f)
    m = jnp.broadcast_to(de[None, :], (f, d))
    base = pe_ref[...] + e0[None, :]
    for k, mref in enumerate(mask_refs):
        a = mref[...].reshape(bb * lc, f).astype(jnp.float32)
        mm = jnp.dot(a, m, preferred_element_type=jnp.float32)
        out_ref[:, k * lc:(k + 1) * lc, :] = (
            mm.reshape(bb, lc, d) + base[None, k * lc:(k + 1) * lc, :]
        )


def kernel(mask, mask_embed):
    B, L, F = mask.shape
    D = mask_embed.shape[1]
    pe = jnp.asarray(_pe_table()[:L])

    BB = 128
    LC = L // NSPLIT
    grid = (B // BB,)
    mask_specs = [
        pl.BlockSpec((BB, LC, F), lambda i, k=k: (i, k, 0)) for k in range(NSPLIT)
    ]
    out = pl.pallas_call(
        _body,
        grid=grid,
        in_specs=mask_specs + [
            pl.BlockSpec((2, D), lambda i: (0, 0)),
            pl.BlockSpec((L, D), lambda i: (0, 0)),
        ],
        out_specs=pl.BlockSpec((BB, L, D), lambda i: (i, 0, 0)),
        out_shape=jax.ShapeDtypeStruct((B, L, D), jnp.float32),
    )(*([mask.astype(jnp.int32)] * NSPLIT), mask_embed, pe)
    return out


# speedup vs baseline: 1.0036x; 1.0036x over previous
import math

import jax
import jax.numpy as jnp
import numpy as np
from jax.experimental import pallas as pl

MAX_LEN = 500
EMBED_DIM = 128
NSPLIT = 1


def _pe_table() -> np.ndarray:
    pe = np.zeros((MAX_LEN, EMBED_DIM), dtype=np.float32)
    position = np.arange(0, MAX_LEN, dtype=np.float32)[:, None]
    div_term = np.exp(
        np.arange(0, EMBED_DIM, 2, dtype=np.float32) * (-math.log(10000.0) / EMBED_DIM)
    )
    pe[:, 0::2] = np.sin(position * div_term)
    pe[:, 1::2] = np.cos(position * div_term)
    return pe


def _body(*refs):
    mask_refs = refs[:NSPLIT]
    emb_ref, pe_ref, out_ref = refs[NSPLIT:]
    bb, lc, f = mask_refs[0].shape
    d = out_ref.shape[-1]
    e0 = emb_ref[0, :]
    de = (emb_ref[1, :] - e0) * (1.0 / f)
    m = jnp.broadcast_to(de[None, :], (f, d))
    base = pe_ref[...] + e0[None, :]
    for k, mref in enumerate(mask_refs):
        a = mref[...].reshape(bb * lc, f).astype(jnp.float32)
        mm = jnp.dot(a, m, preferred_element_type=jnp.float32)
        out_ref[:, k * lc:(k + 1) * lc, :] = (
            mm.reshape(bb, lc, d) + base[None, k * lc:(k + 1) * lc, :]
        )


def kernel(mask, mask_embed):
    B, L, F = mask.shape
    D = mask_embed.shape[1]
    pe = jnp.asarray(_pe_table()[:L])

    BB = 128
    LC = L // NSPLIT
    grid = (B // BB,)
    mask_specs = [
        pl.BlockSpec((BB, LC, F), lambda i, k=k: (i, k, 0)) for k in range(NSPLIT)
    ]
    out = pl.pallas_call(
        _body,
        grid=grid,
        in_specs=mask_specs + [
            pl.BlockSpec((2, D), lambda i: (0, 0)),
            pl.BlockSpec((L, D), lambda i: (0, 0)),
        ],
        out_specs=pl.BlockSpec((BB, L, D), lambda i: (i, 0, 0)),
        out_shape=jax.ShapeDtypeStruct((B, L, D), jnp.float32),
    )(*([mask.astype(jnp.int32)] * NSPLIT), mask_embed, pe)
    return out


# speedup vs baseline: 1.0503x; 1.0465x over previous
import math

import jax
import jax.numpy as jnp
import numpy as np
from jax.experimental import pallas as pl

MAX_LEN = 500
EMBED_DIM = 128


def _pe_table() -> np.ndarray:
    pe = np.zeros((MAX_LEN, EMBED_DIM), dtype=np.float32)
    position = np.arange(0, MAX_LEN, dtype=np.float32)[:, None]
    div_term = np.exp(
        np.arange(0, EMBED_DIM, 2, dtype=np.float32) * (-math.log(10000.0) / EMBED_DIM)
    )
    pe[:, 0::2] = np.sin(position * div_term)
    pe[:, 1::2] = np.cos(position * div_term)
    return pe


def _sum_matrix(L, F, n):
    s = np.zeros((L * F, n), dtype=np.float32)
    rows = np.arange(L * F)
    s[rows, rows // F] = 1.0
    return s


def _body(lc, mask_ref, emb_ref, s_ref, e_ref, pe_ref, out_ref):
    bb, ll, d = out_ref.shape
    f = mask_ref.shape[1] // ll
    e0 = emb_ref[0, :]
    base = pe_ref[...] + e0[None, :]
    a = mask_ref[...].astype(jnp.float32)
    s = jnp.dot(a, s_ref[...], preferred_element_type=jnp.float32)
    for j in range(ll // lc):
        mm = jnp.dot(
            s[:, j * lc:(j + 1) * lc], e_ref[...],
            preferred_element_type=jnp.float32,
        )
        out_ref[:, j * lc:(j + 1) * lc, :] = (
            mm.reshape(bb, lc, d) + base[None, j * lc:(j + 1) * lc, :]
        )


def kernel(mask, mask_embed):
    B, L, F = mask.shape
    D = mask_embed.shape[1]
    pe = jnp.asarray(_pe_table()[:L])

    BB = 128
    LC = 40
    NPAD = 256

    de = (mask_embed[1] - mask_embed[0]) * (1.0 / F)
    emat = jnp.kron(jnp.eye(LC, dtype=jnp.float32), de[None, :])
    smat = jnp.asarray(_sum_matrix(L, F, NPAD))
    mask2 = mask.astype(jnp.int32).reshape(B, L * F)

    import functools
    grid = (B // BB,)
    out = pl.pallas_call(
        functools.partial(_body, LC),
        grid=grid,
        in_specs=[
            pl.BlockSpec((BB, L * F), lambda i: (i, 0)),
            pl.BlockSpec((2, D), lambda i: (0, 0)),
            pl.BlockSpec((L * F, NPAD), lambda i: (0, 0)),
            pl.BlockSpec((LC, LC * D), lambda i: (0, 0)),
            pl.BlockSpec((L, D), lambda i: (0, 0)),
        ],
        out_specs=pl.BlockSpec((BB, L, D), lambda i: (i, 0, 0)),
        out_shape=jax.ShapeDtypeStruct((B, L, D), jnp.float32),
    )(mask2, mask_embed, smat, emat, pe)
    return out
